# Initial kernel scaffold; baseline (speedup 1.0000x reference)
import jax
import jax.numpy as jnp
from jax.experimental import pallas as pl


def kernel(x, edge_index, W_in, b_in, Wg1, bg1, Wg2, bg2, Wg3, bg3, W_out, b_out):
    raise NotImplementedError("write your pallas kernel here")


# speedup vs baseline: 12.6807x; 12.6807x over previous
import functools

import jax
import jax.numpy as jnp
from jax import lax
from jax.experimental import pallas as pl
from jax.experimental.pallas import tpu as pltpu
from jax.experimental.pallas import tpu_sc as plsc

N = 10000
E = 320000
H = 128
NC = 2
NS = 16
NW = NC * NS
EPW = E // NW
CH = 80
STEPS = EPW // CH
NZT = 10
RPT = N // NZT

_mesh = plsc.VectorSubcoreMesh(core_axis_name="c", subcore_axis_name="s")


def _deg_body(dst_hbm, z_hbm, ones_hbm, out_hbm, dst_t, ones_v, acc_sh):
    c = lax.axis_index("c")
    s = lax.axis_index("s")
    wid = s * NC + c

    @pl.when(s < NZT)
    def _():
        pltpu.sync_copy(z_hbm, acc_sh.at[pl.ds(s * RPT, RPT)])

    pltpu.sync_copy(dst_hbm.at[wid], dst_t)
    pltpu.sync_copy(ones_hbm, ones_v)
    plsc.subcore_barrier()

    def body(i, carry):
        pltpu.sync_copy(ones_v, acc_sh.at[dst_t.at[i]], add=True)
        return carry

    lax.fori_loop(0, STEPS, body, 0)
    plsc.subcore_barrier()

    @pl.when(s < NZT)
    def _():
        pltpu.sync_copy(acc_sh.at[pl.ds(s * RPT, RPT)],
                        out_hbm.at[c, pl.ds(s * RPT, RPT)])


_deg_call = pl.kernel(
    _deg_body,
    out_type=jax.ShapeDtypeStruct((NC, N, H), jnp.float32),
    mesh=_mesh,
    scratch_types=[
        pltpu.VMEM((STEPS, CH), jnp.int32),
        pltpu.VMEM((CH, H), jnp.float32),
        pltpu.VMEM_SHARED((N, H), jnp.float32),
    ],
)


def _seg_body(src_hbm, dst_hbm, m_hbm, z_hbm, out_hbm,
              src_t, dst_t, rows_v, acc_sh, sem):
    c = lax.axis_index("c")
    s = lax.axis_index("s")
    wid = s * NC + c

    @pl.when(s < NZT)
    def _():
        pltpu.sync_copy(z_hbm, acc_sh.at[pl.ds(s * RPT, RPT)])

    pltpu.sync_copy(src_hbm.at[wid], src_t)
    pltpu.sync_copy(dst_hbm.at[wid], dst_t)
    plsc.subcore_barrier()

    def body(i, carry):
        pltpu.async_copy(m_hbm.at[src_t.at[i]], rows_v, sem).wait()
        pltpu.sync_copy(rows_v, acc_sh.at[dst_t.at[i]], add=True)
        return carry

    lax.fori_loop(0, STEPS, body, 0)
    plsc.subcore_barrier()

    @pl.when(s < NZT)
    def _():
        pltpu.sync_copy(acc_sh.at[pl.ds(s * RPT, RPT)],
                        out_hbm.at[c, pl.ds(s * RPT, RPT)])


_seg_call = pl.kernel(
    _seg_body,
    out_type=jax.ShapeDtypeStruct((NC, N, H), jnp.float32),
    mesh=_mesh,
    scratch_types=[
        pltpu.VMEM((STEPS, CH), jnp.int32),
        pltpu.VMEM((STEPS, CH), jnp.int32),
        pltpu.VMEM((CH, H), jnp.float32),
        pltpu.VMEM_SHARED((N, H), jnp.float32),
        pltpu.SemaphoreType.DMA,
    ],
)


_GB = 10
_BR = N // _GB


def _dinv_blk(dega_ref, degb_ref):
    deg = dega_ref[:, 0:1] + degb_ref[:, 0:1] + 1.0
    return lax.rsqrt(deg)


def _proj_body(x_ref, win_ref, bin_ref, wg_ref, dega_ref, degb_ref, m_ref):
    dinv = _dinv_blk(dega_ref, degb_ref)
    h = jnp.dot(x_ref[...], win_ref[...],
                preferred_element_type=jnp.float32) + bin_ref[...]
    m_ref[...] = jnp.dot(h, wg_ref[...],
                         preferred_element_type=jnp.float32) * dinv


_proj_call = pl.pallas_call(
    _proj_body,
    grid=(_GB,),
    in_specs=[
        pl.BlockSpec((_BR, H), lambda i: (i, 0)),
        pl.BlockSpec((H, H), lambda i: (0, 0)),
        pl.BlockSpec((1, H), lambda i: (0, 0)),
        pl.BlockSpec((H, H), lambda i: (0, 0)),
        pl.BlockSpec((_BR, H), lambda i: (i, 0)),
        pl.BlockSpec((_BR, H), lambda i: (i, 0)),
    ],
    out_specs=pl.BlockSpec((_BR, H), lambda i: (i, 0)),
    out_shape=jax.ShapeDtypeStruct((N, H), jnp.float32),
)


def _mid_body(sa_ref, sb_ref, mp_ref, dega_ref, degb_ref, b_ref, wg_ref,
              m_ref):
    dinv = _dinv_blk(dega_ref, degb_ref)
    agg = (sa_ref[...] + sb_ref[...] + mp_ref[...]) * dinv
    h = jnp.maximum(agg + b_ref[...], 0.0)
    m_ref[...] = jnp.dot(h, wg_ref[...],
                         preferred_element_type=jnp.float32) * dinv


_mid_call = pl.pallas_call(
    _mid_body,
    grid=(_GB,),
    in_specs=[
        pl.BlockSpec((_BR, H), lambda i: (i, 0)),
        pl.BlockSpec((_BR, H), lambda i: (i, 0)),
        pl.BlockSpec((_BR, H), lambda i: (i, 0)),
        pl.BlockSpec((_BR, H), lambda i: (i, 0)),
        pl.BlockSpec((_BR, H), lambda i: (i, 0)),
        pl.BlockSpec((1, H), lambda i: (0, 0)),
        pl.BlockSpec((H, H), lambda i: (0, 0)),
    ],
    out_specs=pl.BlockSpec((_BR, H), lambda i: (i, 0)),
    out_shape=jax.ShapeDtypeStruct((N, H), jnp.float32),
)


def _fin_body(sa_ref, sb_ref, mp_ref, dega_ref, degb_ref, b_ref, wo_ref,
              bo_ref, out_ref, acc_ref):
    i = pl.program_id(0)

    @pl.when(i == 0)
    def _():
        acc_ref[...] = jnp.zeros_like(acc_ref)

    dinv = _dinv_blk(dega_ref, degb_ref)
    agg = (sa_ref[...] + sb_ref[...] + mp_ref[...]) * dinv
    h = jnp.maximum(agg + b_ref[...], 0.0)
    acc_ref[...] += jnp.sum(h, axis=0, keepdims=True)

    @pl.when(i == _GB - 1)
    def _():
        pooled = acc_ref[...] * (1.0 / N)
        out_ref[...] = (jnp.sum(pooled * wo_ref[...], axis=1, keepdims=True)
                        + bo_ref[...])


_fin_call = pl.pallas_call(
    _fin_body,
    grid=(_GB,),
    in_specs=[
        pl.BlockSpec((_BR, H), lambda i: (i, 0)),
        pl.BlockSpec((_BR, H), lambda i: (i, 0)),
        pl.BlockSpec((_BR, H), lambda i: (i, 0)),
        pl.BlockSpec((_BR, H), lambda i: (i, 0)),
        pl.BlockSpec((_BR, H), lambda i: (i, 0)),
        pl.BlockSpec((1, H), lambda i: (0, 0)),
        pl.BlockSpec((1, H), lambda i: (0, 0)),
        pl.BlockSpec((1, 1), lambda i: (0, 0)),
    ],
    out_specs=pl.BlockSpec((1, 1), lambda i: (0, 0)),
    out_shape=jax.ShapeDtypeStruct((1, 1), jnp.float32),
    scratch_shapes=[pltpu.VMEM((1, H), jnp.float32)],
)


def kernel(x, edge_index, W_in, b_in, Wg1, bg1, Wg2, bg2, Wg3, bg3, W_out,
           b_out):
    src3 = edge_index[0].reshape(NW, STEPS, CH)
    dst3 = edge_index[1].reshape(NW, STEPS, CH)
    z128 = jnp.zeros((RPT, H), jnp.float32)
    ones = jnp.ones((CH, H), jnp.float32)

    deg2 = _deg_call(dst3, z128, ones)
    dega, degb = deg2[0], deg2[1]

    b_in2 = b_in.reshape(1, H)
    m = _proj_call(x, W_in, b_in2, Wg1, dega, degb)

    s = _seg_call(src3, dst3, m, z128)
    m = _mid_call(s[0], s[1], m, dega, degb, bg1.reshape(1, H), Wg2)
    s = _seg_call(src3, dst3, m, z128)
    m = _mid_call(s[0], s[1], m, dega, degb, bg2.reshape(1, H), Wg3)
    s = _seg_call(src3, dst3, m, z128)

    out = _fin_call(s[0], s[1], m, dega, degb, bg3.reshape(1, H),
                    W_out.reshape(1, H), b_out.reshape(1, 1))
    return out


# speedup vs baseline: 19.2790x; 1.5203x over previous
import functools

import jax
import jax.numpy as jnp
from jax import lax
from jax.experimental import pallas as pl
from jax.experimental.pallas import tpu as pltpu
from jax.experimental.pallas import tpu_sc as plsc

N = 10000
E = 320000
H = 128
DW = 16
NC = 2
NS = 16
NW = NC * NS
EPW = E // NW
CH = 125
STEPS = EPW // CH
K = 8
G = STEPS // K
NZT = 10
RPT = N // NZT

_mesh = plsc.VectorSubcoreMesh(core_axis_name="c", subcore_axis_name="s")


def _deg_body(dst_hbm, z_hbm, ones_hbm, out_hbm, dst_t, ones_v, acc_sh):
    c = lax.axis_index("c")
    s = lax.axis_index("s")
    wid = s * NC + c

    @pl.when(s < NZT)
    def _():
        pltpu.sync_copy(z_hbm, acc_sh.at[pl.ds(s * RPT, RPT)])

    pltpu.sync_copy(dst_hbm.at[wid], dst_t)
    pltpu.sync_copy(ones_hbm, ones_v)
    plsc.subcore_barrier()

    def body(i, carry):
        pltpu.sync_copy(ones_v, acc_sh.at[dst_t.at[i]], add=True)
        return carry

    lax.fori_loop(0, STEPS, body, 0)
    plsc.subcore_barrier()

    @pl.when(s < NZT)
    def _():
        pltpu.sync_copy(acc_sh.at[pl.ds(s * RPT, RPT)],
                        out_hbm.at[c, pl.ds(s * RPT, RPT)])


_deg_call = pl.kernel(
    _deg_body,
    out_type=jax.ShapeDtypeStruct((NC, N, DW), jnp.float32),
    mesh=_mesh,
    scratch_types=[
        pltpu.VMEM((STEPS, CH), jnp.int32),
        pltpu.VMEM((CH, DW), jnp.float32),
        pltpu.VMEM_SHARED((N, DW), jnp.float32),
    ],
)


def _seg_body(src_hbm, dst_hbm, m_hbm, z_hbm, out_hbm,
              src_g, dst_g, rows0, rows1, acc_sh, sem0, sem1):
    c = lax.axis_index("c")
    s = lax.axis_index("s")
    wid = s * NC + c

    @pl.when(s < NZT)
    def _():
        pltpu.sync_copy(z_hbm, acc_sh.at[pl.ds(s * RPT, RPT)])

    plsc.subcore_barrier()

    rows = (rows0, rows1)
    sems = (sem0, sem1)

    def grp(g, carry):
        pltpu.sync_copy(src_hbm.at[wid, pl.ds(g * K, K)], src_g)
        pltpu.sync_copy(dst_hbm.at[wid, pl.ds(g * K, K)], dst_g)
        pltpu.async_copy(m_hbm.at[src_g.at[0]], rows0, sem0)
        for j in range(K):
            b = j % 2
            if j + 1 < K:
                pltpu.async_copy(m_hbm.at[src_g.at[j + 1]], rows[1 - b],
                                 sems[1 - b])
            pltpu.make_async_copy(m_hbm.at[src_g.at[j]], rows[b],
                                  sems[b]).wait()
            pltpu.sync_copy(rows[b], acc_sh.at[dst_g.at[j]], add=True)
        return carry

    lax.fori_loop(0, G, grp, 0)
    plsc.subcore_barrier()

    @pl.when(s < NZT)
    def _():
        pltpu.sync_copy(acc_sh.at[pl.ds(s * RPT, RPT)],
                        out_hbm.at[c, pl.ds(s * RPT, RPT)])


_seg_call = pl.kernel(
    _seg_body,
    out_type=jax.ShapeDtypeStruct((NC, N, H), jnp.float32),
    mesh=_mesh,
    scratch_types=[
        pltpu.VMEM((K, CH), jnp.int32),
        pltpu.VMEM((K, CH), jnp.int32),
        pltpu.VMEM((CH, H), jnp.float32),
        pltpu.VMEM((CH, H), jnp.float32),
        pltpu.VMEM_SHARED((N, H), jnp.float32),
        pltpu.SemaphoreType.DMA,
        pltpu.SemaphoreType.DMA,
    ],
)


_GB = 10
_BR = N // _GB


def _dinv_blk(dega_ref, degb_ref):
    deg = dega_ref[:, 0:1] + degb_ref[:, 0:1] + 1.0
    return lax.rsqrt(deg)


def _proj_body(x_ref, win_ref, bin_ref, wg_ref, dega_ref, degb_ref, m_ref):
    dinv = _dinv_blk(dega_ref, degb_ref)
    h = jnp.dot(x_ref[...], win_ref[...],
                preferred_element_type=jnp.float32) + bin_ref[...]
    m_ref[...] = jnp.dot(h, wg_ref[...],
                         preferred_element_type=jnp.float32) * dinv


_proj_call = pl.pallas_call(
    _proj_body,
    grid=(_GB,),
    in_specs=[
        pl.BlockSpec((_BR, H), lambda i: (i, 0)),
        pl.BlockSpec((H, H), lambda i: (0, 0)),
        pl.BlockSpec((1, H), lambda i: (0, 0)),
        pl.BlockSpec((H, H), lambda i: (0, 0)),
        pl.BlockSpec((_BR, DW), lambda i: (i, 0)),
        pl.BlockSpec((_BR, DW), lambda i: (i, 0)),
    ],
    out_specs=pl.BlockSpec((_BR, H), lambda i: (i, 0)),
    out_shape=jax.ShapeDtypeStruct((N, H), jnp.float32),
)


def _mid_body(sa_ref, sb_ref, mp_ref, dega_ref, degb_ref, b_ref, wg_ref,
              m_ref):
    dinv = _dinv_blk(dega_ref, degb_ref)
    agg = (sa_ref[...] + sb_ref[...] + mp_ref[...]) * dinv
    h = jnp.maximum(agg + b_ref[...], 0.0)
    m_ref[...] = jnp.dot(h, wg_ref[...],
                         preferred_element_type=jnp.float32) * dinv


_mid_call = pl.pallas_call(
    _mid_body,
    grid=(_GB,),
    in_specs=[
        pl.BlockSpec((_BR, H), lambda i: (i, 0)),
        pl.BlockSpec((_BR, H), lambda i: (i, 0)),
        pl.BlockSpec((_BR, H), lambda i: (i, 0)),
        pl.BlockSpec((_BR, DW), lambda i: (i, 0)),
        pl.BlockSpec((_BR, DW), lambda i: (i, 0)),
        pl.BlockSpec((1, H), lambda i: (0, 0)),
        pl.BlockSpec((H, H), lambda i: (0, 0)),
    ],
    out_specs=pl.BlockSpec((_BR, H), lambda i: (i, 0)),
    out_shape=jax.ShapeDtypeStruct((N, H), jnp.float32),
)


def _fin_body(sa_ref, sb_ref, mp_ref, dega_ref, degb_ref, b_ref, wo_ref,
              bo_ref, out_ref, acc_ref):
    i = pl.program_id(0)

    @pl.when(i == 0)
    def _():
        acc_ref[...] = jnp.zeros_like(acc_ref)

    dinv = _dinv_blk(dega_ref, degb_ref)
    agg = (sa_ref[...] + sb_ref[...] + mp_ref[...]) * dinv
    h = jnp.maximum(agg + b_ref[...], 0.0)
    acc_ref[...] += jnp.sum(h, axis=0, keepdims=True)

    @pl.when(i == _GB - 1)
    def _():
        pooled = acc_ref[...] * (1.0 / N)
        out_ref[...] = (jnp.sum(pooled * wo_ref[...], axis=1, keepdims=True)
                        + bo_ref[...])


_fin_call = pl.pallas_call(
    _fin_body,
    grid=(_GB,),
    in_specs=[
        pl.BlockSpec((_BR, H), lambda i: (i, 0)),
        pl.BlockSpec((_BR, H), lambda i: (i, 0)),
        pl.BlockSpec((_BR, H), lambda i: (i, 0)),
        pl.BlockSpec((_BR, DW), lambda i: (i, 0)),
        pl.BlockSpec((_BR, DW), lambda i: (i, 0)),
        pl.BlockSpec((1, H), lambda i: (0, 0)),
        pl.BlockSpec((1, H), lambda i: (0, 0)),
        pl.BlockSpec((1, 1), lambda i: (0, 0)),
    ],
    out_specs=pl.BlockSpec((1, 1), lambda i: (0, 0)),
    out_shape=jax.ShapeDtypeStruct((1, 1), jnp.float32),
    scratch_shapes=[pltpu.VMEM((1, H), jnp.float32)],
)


def kernel(x, edge_index, W_in, b_in, Wg1, bg1, Wg2, bg2, Wg3, bg3, W_out,
           b_out):
    src3 = edge_index[0].reshape(NW, STEPS, CH)
    dst3 = edge_index[1].reshape(NW, STEPS, CH)
    z16 = jnp.zeros((RPT, DW), jnp.float32)
    z128 = jnp.zeros((RPT, H), jnp.float32)
    ones = jnp.ones((CH, DW), jnp.float32)

    deg2 = _deg_call(dst3, z16, ones)
    dega, degb = deg2[0], deg2[1]

    b_in2 = b_in.reshape(1, H)
    m = _proj_call(x, W_in, b_in2, Wg1, dega, degb)

    s = _seg_call(src3, dst3, m, z128)
    m = _mid_call(s[0], s[1], m, dega, degb, bg1.reshape(1, H), Wg2)
    s = _seg_call(src3, dst3, m, z128)
    m = _mid_call(s[0], s[1], m, dega, degb, bg2.reshape(1, H), Wg3)
    s = _seg_call(src3, dst3, m, z128)

    out = _fin_call(s[0], s[1], m, dega, degb, bg3.reshape(1, H),
                    W_out.reshape(1, H), b_out.reshape(1, 1))
    return out
